# baseline (speedup 1.0000x reference)
import functools

import jax
import jax.numpy as jnp
from jax import lax
from jax.experimental import pallas as pl
from jax.experimental.pallas import tpu as pltpu
from jax.experimental.pallas import tpu_sc as plsc

_HIDDEN = 128
_N_NODES = 10000
_N_EDGES = 320000

_NC = 2
_NS = 16
_NW = _NC * _NS
_EPW = _N_EDGES // _NW
_BB = 200
_NBLK = _EPW // _BB
_CHUNKS = ((0, 104), (104, 96))
_GROUPS = (_BB + 15) // 16
_HW = _HIDDEN // 2
_DEPTH = 2
_STAGE = 632
_NPAD = _STAGE * _NS


def _zw_body(z_ref, w_ref, out_ref):
    w = w_ref[...]
    out_ref[...] = (
        jnp.dot(z_ref[...], w, preferred_element_type=jnp.float32)
        + lax.dot_general(z_ref[...], w, (((1,), (1,)), ((), ())),
                          preferred_element_type=jnp.float32)
    )


def _compute_zw(z, W):
    return pl.pallas_call(
        _zw_body,
        out_shape=jax.ShapeDtypeStruct((_N_NODES, _HIDDEN), jnp.float32),
    )(z, W)


def _sc_body(z_hbm, zw_hbm, src_hbm, dst_hbm, out_hbm,
             src_v, dst_v, a_v, b_v, o_v, zw_sh, *sems):
    sid = lax.axis_index("s")
    wid = sid * _NC + lax.axis_index("c")
    base = wid * _EPW
    sem_g = sems[:_DEPTH]
    sem_h = sems[_DEPTH:2 * _DEPTH]
    sem_w = sems[2 * _DEPTH:]
    lane = lax.iota(jnp.int32, 16)

    pltpu.sync_copy(src_hbm.at[pl.ds(base, _EPW)], src_v)
    pltpu.sync_copy(dst_hbm.at[pl.ds(base, _EPW)], dst_v)

    slab = pl.multiple_of(sid * _STAGE, 8)
    for off, n in ((0, 312), (312, 320)):
        pltpu.sync_copy(zw_hbm.at[pl.ds(slab + off, n)], a_v.at[pl.ds(0, n)])
        pltpu.sync_copy(a_v.at[pl.ds(0, n)], zw_sh.at[pl.ds(slab + off, n)])
    plsc.subcore_barrier()

    def gather_descs(i, s):
        descs = []
        for tab, idxbuf, rowbuf, sem in ((z_hbm, src_v, a_v, sem_g),
                                         (zw_sh, dst_v, b_v, sem_h)):
            for off, n in _CHUNKS:
                descs.append(pltpu.make_async_copy(
                    tab.at[idxbuf.at[pl.ds(i * _BB + off, n)]],
                    rowbuf.at[pl.ds(s * _BB + off, n)],
                    sem[s]))
        return descs

    def start_gathers(i, s):
        for d in gather_descs(i, s):
            d.start()

    def wait_gathers(i, s):
        for d in gather_descs(i, s):
            d.wait()

    def wb_desc(i, s):
        return pltpu.make_async_copy(
            o_v.at[pl.ds(s * (_GROUPS * 16), _BB)],
            out_hbm.at[pl.ds(base + i * _BB, _BB)],
            sem_w[s])

    def compute(i, s):
        def grp(g, c):
            row0 = s * _BB + g * 16

            def edge_k(k, vec):
                e = row0 + k
                prods = []
                for j in range(_HW // 16):
                    aw = plsc.bitcast(a_v[e, pl.ds(j * 16, 16)], jnp.bfloat16)
                    bw = plsc.bitcast(b_v[e, pl.ds(j * 16, 16)], jnp.bfloat16)
                    a0, a1 = plsc.unpack(aw, format=plsc.PackFormat.INTERLEAVED)
                    b0, b1 = plsc.unpack(bw, format=plsc.PackFormat.INTERLEAVED)
                    prods.append(a0 * b0)
                    prods.append(a1 * b1)
                while len(prods) > 1:
                    prods = [prods[p] + prods[p + 1]
                             for p in range(0, len(prods), 2)]
                return jnp.where(lane == k, jnp.sum(prods[0]), vec)

            vec = lax.fori_loop(0, 16, edge_k, jnp.zeros((16,), jnp.float32),
                                unroll=4)
            o_v[pl.ds(s * (_GROUPS * 16) + g * 16, 16)] = (
                1.0 / (1.0 + jnp.exp(-vec)))
            return c

        lax.fori_loop(0, _GROUPS, grp, 0)

    def section(i, s):
        @pl.when(i < _NBLK)
        def _():
            @pl.when(i + _DEPTH - 1 < _NBLK)
            def _():
                start_gathers(i + _DEPTH - 1, (s + _DEPTH - 1) % _DEPTH)

            wait_gathers(i, s)

            @pl.when(i >= _DEPTH)
            def _():
                wb_desc(i, s).wait()

            compute(i, s)
            wb_desc(i, s).start()

    for s in range(_DEPTH - 1):
        start_gathers(s, s)

    def body(t, c):
        for s in range(_DEPTH):
            section(t * _DEPTH + s, s)
        return c

    lax.fori_loop(0, (_NBLK + _DEPTH - 1) // _DEPTH, body, 0)
    for s in range(_DEPTH):
        wb_desc(_NBLK - _DEPTH + s, s).wait()


@functools.partial(
    pl.kernel,
    out_type=jax.ShapeDtypeStruct((_N_EDGES,), jnp.float32),
    mesh=plsc.VectorSubcoreMesh(core_axis_name="c", subcore_axis_name="s"),
    compiler_params=pltpu.CompilerParams(needs_layout_passes=False,
                                         use_tc_tiling_on_sc=False),
    scratch_types=[
        pltpu.VMEM_SHARED((_NPAD, _HW), jnp.float32),
        pltpu.VMEM((_EPW,), jnp.int32),
        pltpu.VMEM((_EPW,), jnp.int32),
        pltpu.VMEM((_DEPTH * _BB + 8, _HW), jnp.float32),
        pltpu.VMEM((_DEPTH * _BB + 8, _HW), jnp.float32),
        pltpu.VMEM((_DEPTH * _GROUPS * 16,), jnp.float32),
    ] + [pltpu.SemaphoreType.DMA] * (3 * _DEPTH),
)
def _sc_score(z_hbm, zw_hbm, src_hbm, dst_hbm, out_hbm,
              zw_sh, src_v, dst_v, a_v, b_v, o_v, *sems):
    _sc_body(z_hbm, zw_hbm, src_hbm, dst_hbm, out_hbm,
             src_v, dst_v, a_v, b_v, o_v, zw_sh, *sems)


def _pack_bf16(t):
    tb = t.astype(jnp.bfloat16).reshape(t.shape[0], _HW, 2)
    packed = lax.bitcast_convert_type(tb, jnp.float32)
    return jnp.pad(packed, ((0, _NPAD - t.shape[0]), (0, 0)))


def kernel(z, edge_index, W):
    zw = _compute_zw(z, W)
    src = edge_index[0].astype(jnp.int32)
    dst = edge_index[1].astype(jnp.int32)
    return _sc_score(_pack_bf16(z), _pack_bf16(zw), src, dst)


# speedup vs baseline: 1.1592x; 1.1592x over previous
import functools

import jax
import jax.numpy as jnp
from jax import lax
from jax.experimental import pallas as pl
from jax.experimental.pallas import tpu as pltpu
from jax.experimental.pallas import tpu_sc as plsc

_HIDDEN = 128
_N_NODES = 10000
_N_EDGES = 320000

_NC = 2
_NS = 16
_NW = _NC * _NS
_EPW = _N_EDGES // _NW
_BB = 200
_NBLK = _EPW // _BB
_CHUNKS = ((0, 104), (104, 96))
_GROUPS = (_BB + 15) // 16
_DEPTH = 2


def _zw_body(z_ref, w_ref, out_ref):
    w = w_ref[...]
    out_ref[...] = (
        jnp.dot(z_ref[...], w, preferred_element_type=jnp.float32)
        + lax.dot_general(z_ref[...], w, (((1,), (1,)), ((), ())),
                          preferred_element_type=jnp.float32)
    )


def _compute_zw(z, W):
    return pl.pallas_call(
        _zw_body,
        out_shape=jax.ShapeDtypeStruct((_N_NODES, _HIDDEN), jnp.float32),
    )(z, W)


def _sc_body(z_hbm, zw_hbm, src_hbm, dst_hbm, out_hbm,
             src_v, dst_v, a_v, b_v, o_v, *sems):
    wid = lax.axis_index("s") * _NC + lax.axis_index("c")
    base = wid * _EPW
    sem_g = sems[:_DEPTH]
    sem_h = sems[_DEPTH:2 * _DEPTH]
    sem_w = sems[2 * _DEPTH:]
    lane = lax.iota(jnp.int32, 16)

    pltpu.sync_copy(src_hbm.at[pl.ds(base, _EPW)], src_v)
    pltpu.sync_copy(dst_hbm.at[pl.ds(base, _EPW)], dst_v)

    def gather_descs(i, s):
        descs = []
        for tab, idxbuf, rowbuf, sem in ((z_hbm, src_v, a_v, sem_g),
                                         (zw_hbm, dst_v, b_v, sem_h)):
            for off, n in _CHUNKS:
                descs.append(pltpu.make_async_copy(
                    tab.at[idxbuf.at[pl.ds(i * _BB + off, n)]],
                    rowbuf.at[pl.ds(s * _BB + off, n)],
                    sem[s]))
        return descs

    def start_gathers(i, s):
        for d in gather_descs(i, s):
            d.start()

    def wait_gathers(i, s):
        for d in gather_descs(i, s):
            d.wait()

    def wb_desc(i, s):
        return pltpu.make_async_copy(
            o_v.at[pl.ds(s * (_GROUPS * 16), _BB)],
            out_hbm.at[pl.ds(base + i * _BB, _BB)],
            sem_w[s])

    def compute(i, s):
        def grp(g, c):
            row0 = s * _BB + g * 16

            def edge_k(k, vec):
                e = row0 + k
                prods = [a_v[e, pl.ds(j * 16, 16)] * b_v[e, pl.ds(j * 16, 16)]
                         for j in range(_HIDDEN // 16)]
                while len(prods) > 1:
                    prods = [prods[p] + prods[p + 1]
                             for p in range(0, len(prods), 2)]
                return jnp.where(lane == k, jnp.sum(prods[0]), vec)

            vec = lax.fori_loop(0, 16, edge_k, jnp.zeros((16,), jnp.float32),
                                unroll=4)
            o_v[pl.ds(s * (_GROUPS * 16) + g * 16, 16)] = (
                1.0 / (1.0 + jnp.exp(-vec)))
            return c

        lax.fori_loop(0, _GROUPS, grp, 0)

    def section(i, s):
        @pl.when(i < _NBLK)
        def _():
            @pl.when(i + _DEPTH - 1 < _NBLK)
            def _():
                start_gathers(i + _DEPTH - 1, (s + _DEPTH - 1) % _DEPTH)

            wait_gathers(i, s)

            @pl.when(i >= _DEPTH)
            def _():
                wb_desc(i, s).wait()

            compute(i, s)
            wb_desc(i, s).start()

    for s in range(_DEPTH - 1):
        start_gathers(s, s)

    def body(t, c):
        for s in range(_DEPTH):
            section(t * _DEPTH + s, s)
        return c

    lax.fori_loop(0, (_NBLK + _DEPTH - 1) // _DEPTH, body, 0)
    for s in range(_DEPTH):
        wb_desc(_NBLK - _DEPTH + s, s).wait()


@functools.partial(
    pl.kernel,
    out_type=jax.ShapeDtypeStruct((_N_EDGES,), jnp.float32),
    mesh=plsc.VectorSubcoreMesh(core_axis_name="c", subcore_axis_name="s"),
    compiler_params=pltpu.CompilerParams(needs_layout_passes=False),
    scratch_types=[
        pltpu.VMEM((_EPW,), jnp.int32),
        pltpu.VMEM((_EPW,), jnp.int32),
        pltpu.VMEM((_DEPTH * _BB + 8, _HIDDEN), jnp.float32),
        pltpu.VMEM((_DEPTH * _BB + 8, _HIDDEN), jnp.float32),
        pltpu.VMEM((_DEPTH * _GROUPS * 16,), jnp.float32),
    ] + [pltpu.SemaphoreType.DMA] * (3 * _DEPTH),
)
def _sc_score(z_hbm, zw_hbm, src_hbm, dst_hbm, out_hbm,
              src_v, dst_v, a_v, b_v, o_v, *sems):
    _sc_body(z_hbm, zw_hbm, src_hbm, dst_hbm, out_hbm,
             src_v, dst_v, a_v, b_v, o_v, *sems)


def kernel(z, edge_index, W):
    zw = _compute_zw(z, W)
    src = edge_index[0].astype(jnp.int32)
    dst = edge_index[1].astype(jnp.int32)
    return _sc_score(z, zw, src, dst)
